# Initial kernel scaffold; baseline (speedup 1.0000x reference)
import jax
import jax.numpy as jnp
from jax.experimental import pallas as pl


def kernel(q, k, v):
    raise NotImplementedError("write your pallas kernel here")


# speedup vs baseline: 1.0738x; 1.0738x over previous
import jax
import jax.numpy as jnp
from jax import lax
from jax.experimental import pallas as pl


def _stages(n):
    out = []
    kk = 2
    while kk <= n:
        j = kk // 2
        while j >= 1:
            out.append((kk, j))
            j //= 2
        kk *= 2
    return out


def _partner(a, j):
    n, c = a.shape
    r = a.reshape(n // (2 * j), 2 * j, c)
    p = jnp.concatenate([r[:, j:], r[:, :j]], axis=1)
    return p.reshape(n, c)


def _bitonic(key, idx, payloads):
    n, c = key.shape
    row = lax.broadcasted_iota(jnp.int32, (n, c), 0)
    arrs = [key] + ([idx] if idx is not None else []) + list(payloads)
    for kk, j in _stages(n):
        partners = [_partner(a, j) for a in arrs]
        pk = partners[0]
        if idx is not None:
            pidx = partners[1]
            t = (arrs[0] > pk) | ((arrs[0] == pk) & (arrs[1] > pidx))
        else:
            t = arrs[0] > pk
        is_lo = (row & j) == 0
        asc = (row & kk) == 0
        take = (t == is_lo) == asc
        arrs = [jnp.where(take, p, a) for a, p in zip(arrs, partners)]
    return arrs


def _sort_kernel(q_ref, k_ref, v_ref, o_ref):
    q = q_ref[0]
    k = k_ref[0]
    v = v_ref[0]
    n, c = q.shape
    idx = lax.broadcasted_iota(jnp.int32, (n, c), 0)
    _, _, w = _bitonic(k, idx, [v])
    _, q_idx = _bitonic(q, idx, [])
    _, out = _bitonic(q_idx, None, [w])
    o_ref[0] = out


def _pack(x):
    b, h, n, c = x.shape
    g = b * h // 2
    return x.reshape(g, 2, n, c).transpose(0, 2, 1, 3).reshape(g, n, 2 * c)


def _unpack(y, b, h, c):
    g, n, c2 = y.shape
    return y.reshape(g, n, 2, c).transpose(0, 2, 1, 3).reshape(b, h, n, c)


def kernel(q, k, v):
    b, h, n, c = q.shape
    qp, kp, vp = _pack(q), _pack(k), _pack(v)
    g, _, lanes = qp.shape
    out = pl.pallas_call(
        _sort_kernel,
        grid=(g,),
        in_specs=[pl.BlockSpec((1, n, lanes), lambda i: (i, 0, 0))] * 3,
        out_specs=pl.BlockSpec((1, n, lanes), lambda i: (i, 0, 0)),
        out_shape=jax.ShapeDtypeStruct((g, n, lanes), jnp.float32),
    )(qp, kp, vp)
    o = _unpack(out, b, h, c)
    return (o, o)
